# baseline (speedup 1.0000x reference)
import functools

import jax
import jax.numpy as jnp
from jax import lax
from jax.experimental import pallas as pl
from jax.experimental.pallas import tpu as pltpu
from jax.experimental.pallas import tpu_sc as plsc

EMBED_DIM = 64
NUM_CORES = 2
NUM_SUBCORES = 16
NUM_WORKERS = NUM_CORES * NUM_SUBCORES
CHUNK = 1024


def kernel(x, weight):
    batch, hist = x.shape
    num_indices = batch * hist
    idx = x.reshape(num_indices).astype(jnp.int32)
    per_worker = num_indices // NUM_WORKERS

    mesh = plsc.VectorSubcoreMesh(core_axis_name="c", subcore_axis_name="s")

    @functools.partial(
        pl.kernel,
        mesh=mesh,
        out_type=jax.ShapeDtypeStruct((num_indices, EMBED_DIM), weight.dtype),
        scratch_types=[
            pltpu.VMEM((CHUNK,), jnp.int32),
            pltpu.VMEM((CHUNK, EMBED_DIM), jnp.float32),
            pltpu.SemaphoreType.DMA,
        ],
        compiler_params=pltpu.CompilerParams(use_tc_tiling_on_sc=False),
    )
    def gather_kernel(idx_hbm, table_hbm, out_hbm, idx_v, rows_v, sem):
        wid = lax.axis_index("s") * NUM_CORES + lax.axis_index("c")
        base = wid * per_worker

        @pl.loop(0, per_worker, step=CHUNK)
        def _(off):
            pltpu.sync_copy(idx_hbm.at[pl.ds(base + off, CHUNK)], idx_v)
            pltpu.async_copy(table_hbm.at[idx_v], rows_v, sem).wait()
            pltpu.sync_copy(rows_v, out_hbm.at[pl.ds(base + off, CHUNK)])

    out = gather_kernel(idx, weight)
    return out.reshape(batch, hist, EMBED_DIM)


# speedup vs baseline: 1.0036x; 1.0036x over previous
import functools

import jax
import jax.numpy as jnp
from jax import lax
from jax.experimental import pallas as pl
from jax.experimental.pallas import tpu as pltpu
from jax.experimental.pallas import tpu_sc as plsc

EMBED_DIM = 64
NUM_CORES = 2
NUM_SUBCORES = 16
NUM_WORKERS = NUM_CORES * NUM_SUBCORES
CHUNK = 512


def kernel(x, weight):
    batch, hist = x.shape
    num_indices = batch * hist
    idx = x.reshape(num_indices).astype(jnp.int32)
    per_worker = num_indices // NUM_WORKERS
    n_chunks = per_worker // CHUNK

    mesh = plsc.VectorSubcoreMesh(core_axis_name="c", subcore_axis_name="s")

    @functools.partial(
        pl.kernel,
        mesh=mesh,
        out_type=jax.ShapeDtypeStruct((num_indices, EMBED_DIM), weight.dtype),
        scratch_types=[
            pltpu.VMEM((per_worker,), jnp.int32),
            pltpu.VMEM((CHUNK, EMBED_DIM), jnp.float32),
            pltpu.VMEM((CHUNK, EMBED_DIM), jnp.float32),
            pltpu.SemaphoreType.DMA,
            pltpu.SemaphoreType.DMA,
            pltpu.SemaphoreType.DMA,
            pltpu.SemaphoreType.DMA,
        ],
        compiler_params=pltpu.CompilerParams(use_tc_tiling_on_sc=False),
    )
    def gather_kernel(idx_hbm, table_hbm, out_hbm, idx_v, r0, r1,
                      g0, g1, w0, w1):
        wid = lax.axis_index("s") * NUM_CORES + lax.axis_index("c")
        base = wid * per_worker
        rbufs = (r0, r1)
        gsems = (g0, g1)
        wsems = (w0, w1)

        pltpu.sync_copy(idx_hbm.at[pl.ds(base, per_worker)], idx_v)

        def idx_slice(c):
            return idx_v.at[pl.ds(c * CHUNK, CHUNK)]

        def out_slice(c):
            return out_hbm.at[pl.ds(base + c * CHUNK, CHUNK)]

        pltpu.async_copy(table_hbm.at[idx_slice(0)], rbufs[0], gsems[0])

        @pl.loop(0, n_chunks, step=2)
        def _(k):
            for b in (0, 1):
                c = k + b
                pltpu.make_async_copy(table_hbm.at[idx_slice(c)],
                                      rbufs[b], gsems[b]).wait()
                pltpu.async_copy(rbufs[b], out_slice(c), wsems[b])

                @pl.when(c + 1 < n_chunks)
                def _():
                    @pl.when(c >= 1)
                    def _():
                        pltpu.make_async_copy(rbufs[1 - b], out_slice(c - 1),
                                              wsems[1 - b]).wait()

                    pltpu.async_copy(table_hbm.at[idx_slice(c + 1)],
                                     rbufs[1 - b], gsems[1 - b])

        pltpu.make_async_copy(rbufs[0], out_slice(n_chunks - 2),
                              wsems[0]).wait()
        pltpu.make_async_copy(rbufs[1], out_slice(n_chunks - 1),
                              wsems[1]).wait()

    out = gather_kernel(idx, weight)
    return out.reshape(batch, hist, EMBED_DIM)


# speedup vs baseline: 1.0058x; 1.0022x over previous
import functools

import jax
import jax.numpy as jnp
from jax import lax
from jax.experimental import pallas as pl
from jax.experimental.pallas import tpu as pltpu
from jax.experimental.pallas import tpu_sc as plsc

EMBED_DIM = 64
NUM_CORES = 2
NUM_SUBCORES = 16
NUM_WORKERS = NUM_CORES * NUM_SUBCORES
CHUNK = 256
NBUF = 4


def kernel(x, weight):
    batch, hist = x.shape
    num_indices = batch * hist
    idx = x.reshape(num_indices).astype(jnp.int32)
    per_worker = num_indices // NUM_WORKERS
    n_chunks = per_worker // CHUNK

    mesh = plsc.VectorSubcoreMesh(core_axis_name="c", subcore_axis_name="s")

    row_buf = pltpu.VMEM((CHUNK, EMBED_DIM), jnp.float32)

    @functools.partial(
        pl.kernel,
        mesh=mesh,
        out_type=jax.ShapeDtypeStruct((num_indices, EMBED_DIM), weight.dtype),
        scratch_types=[
            pltpu.VMEM((per_worker,), jnp.int32),
            *([row_buf] * NBUF),
            *([pltpu.SemaphoreType.DMA] * (2 * NBUF)),
        ],
        compiler_params=pltpu.CompilerParams(use_tc_tiling_on_sc=False),
    )
    def gather_kernel(idx_hbm, table_hbm, out_hbm, idx_v, *bufs_sems):
        rbufs = bufs_sems[:NBUF]
        gsems = bufs_sems[NBUF:2 * NBUF]
        wsems = bufs_sems[2 * NBUF:]

        wid = lax.axis_index("s") * NUM_CORES + lax.axis_index("c")
        base = wid * per_worker

        pltpu.sync_copy(idx_hbm.at[pl.ds(base, per_worker)], idx_v)

        def idx_slice(c):
            return idx_v.at[pl.ds(c * CHUNK, CHUNK)]

        def out_slice(c):
            return out_hbm.at[pl.ds(base + c * CHUNK, CHUNK)]

        for b in range(NBUF - 1):
            pltpu.async_copy(table_hbm.at[idx_slice(b)], rbufs[b], gsems[b])

        @pl.loop(0, n_chunks, step=NBUF)
        def _(k):
            for b in range(NBUF):
                c = k + b
                nxt = c + NBUF - 1
                nb = (b + NBUF - 1) % NBUF
                pltpu.make_async_copy(table_hbm.at[idx_slice(c)],
                                      rbufs[b], gsems[b]).wait()
                pltpu.async_copy(rbufs[b], out_slice(c), wsems[b])

                @pl.when(nxt < n_chunks)
                def _():
                    @pl.when(c >= 1)
                    def _():
                        pltpu.make_async_copy(rbufs[nb], out_slice(c - 1),
                                              wsems[nb]).wait()

                    pltpu.async_copy(table_hbm.at[idx_slice(nxt)],
                                     rbufs[nb], gsems[nb])

        for j in range(NBUF):
            c = n_chunks - NBUF + j
            pltpu.make_async_copy(rbufs[c % NBUF], out_slice(c),
                                  wsems[c % NBUF]).wait()

    out = gather_kernel(idx, weight)
    return out.reshape(batch, hist, EMBED_DIM)


# speedup vs baseline: 1.0454x; 1.0394x over previous
import functools

import jax
import jax.numpy as jnp
from jax import lax
from jax.experimental import pallas as pl
from jax.experimental.pallas import tpu as pltpu
from jax.experimental.pallas import tpu_sc as plsc

EMBED_DIM = 64
PAD_DIM = 128
NUM_CORES = 2
NUM_SUBCORES = 16
NUM_WORKERS = NUM_CORES * NUM_SUBCORES
CHUNK = 160
NBUF = 4


def kernel(x, weight):
    batch, hist = x.shape
    num_indices = batch * hist
    idx = x.reshape(num_indices).astype(jnp.int32)
    per_worker = num_indices // NUM_WORKERS
    n_chunks = per_worker // CHUNK
    w128 = jnp.pad(weight, ((0, 0), (0, PAD_DIM - EMBED_DIM)))

    mesh = plsc.VectorSubcoreMesh(core_axis_name="c", subcore_axis_name="s")

    row_buf = pltpu.VMEM((CHUNK, PAD_DIM), jnp.float32)

    @functools.partial(
        pl.kernel,
        mesh=mesh,
        out_type=jax.ShapeDtypeStruct((num_indices, EMBED_DIM), weight.dtype),
        scratch_types=[
            pltpu.VMEM((per_worker,), jnp.int32),
            *([row_buf] * NBUF),
            *([pltpu.SemaphoreType.DMA] * (2 * NBUF)),
        ],
        compiler_params=pltpu.CompilerParams(use_tc_tiling_on_sc=False),
    )
    def gather_kernel(idx_hbm, table_hbm, out_hbm, idx_v, *bufs_sems):
        rbufs = bufs_sems[:NBUF]
        gsems = bufs_sems[NBUF:2 * NBUF]
        wsems = bufs_sems[2 * NBUF:]

        wid = lax.axis_index("s") * NUM_CORES + lax.axis_index("c")
        base = wid * per_worker

        pltpu.sync_copy(idx_hbm.at[pl.ds(base, per_worker)], idx_v)

        def idx_slice(c):
            return idx_v.at[pl.ds(c * CHUNK, CHUNK)]

        def out_slice(c):
            return out_hbm.at[pl.ds(base + c * CHUNK, CHUNK)]

        def row_half(b):
            return rbufs[b].at[:, pl.ds(0, EMBED_DIM)]

        for b in range(NBUF - 1):
            pltpu.async_copy(table_hbm.at[idx_slice(b)], rbufs[b], gsems[b])

        @pl.loop(0, n_chunks, step=NBUF)
        def _(k):
            for b in range(NBUF):
                c = k + b
                nxt = c + NBUF - 1
                nb = (b + NBUF - 1) % NBUF
                pltpu.make_async_copy(table_hbm.at[idx_slice(c)],
                                      rbufs[b], gsems[b]).wait()
                pltpu.async_copy(row_half(b), out_slice(c), wsems[b])

                @pl.when(nxt < n_chunks)
                def _():
                    @pl.when(c >= 1)
                    def _():
                        pltpu.make_async_copy(row_half(nb), out_slice(c - 1),
                                              wsems[nb]).wait()

                    pltpu.async_copy(table_hbm.at[idx_slice(nxt)],
                                     rbufs[nb], gsems[nb])

        for j in range(NBUF):
            c = n_chunks - NBUF + j
            pltpu.make_async_copy(row_half(c % NBUF), out_slice(c),
                                  wsems[c % NBUF]).wait()

    out = gather_kernel(idx, w128)
    return out.reshape(batch, hist, EMBED_DIM)


# speedup vs baseline: 1.0457x; 1.0003x over previous
import functools

import jax
import jax.numpy as jnp
from jax import lax
from jax.experimental import pallas as pl
from jax.experimental.pallas import tpu as pltpu
from jax.experimental.pallas import tpu_sc as plsc

EMBED_DIM = 64
PAD_DIM = 128
NUM_CORES = 2
NUM_SUBCORES = 16
NUM_WORKERS = NUM_CORES * NUM_SUBCORES
CHUNK = 160
NBUF = 4


def kernel(x, weight):
    batch, hist = x.shape
    num_indices = batch * hist
    idx = x.reshape(num_indices).astype(jnp.int32)
    per_worker = num_indices // NUM_WORKERS
    rows_per_worker = batch // NUM_WORKERS
    rows_per_chunk = CHUNK // hist
    n_chunks = per_worker // CHUNK
    w128 = jnp.pad(weight, ((0, 0), (0, PAD_DIM - EMBED_DIM)))

    mesh = plsc.VectorSubcoreMesh(core_axis_name="c", subcore_axis_name="s")

    row_buf = pltpu.VMEM((CHUNK, PAD_DIM), jnp.float32)

    @functools.partial(
        pl.kernel,
        mesh=mesh,
        out_type=jax.ShapeDtypeStruct((batch, hist, EMBED_DIM), weight.dtype),
        scratch_types=[
            pltpu.VMEM((per_worker,), jnp.int32),
            *([row_buf] * NBUF),
            *([pltpu.SemaphoreType.DMA] * (2 * NBUF)),
        ],
        compiler_params=pltpu.CompilerParams(use_tc_tiling_on_sc=False),
    )
    def gather_kernel(idx_hbm, table_hbm, out_hbm, idx_v, *bufs_sems):
        rbufs = bufs_sems[:NBUF]
        gsems = bufs_sems[NBUF:2 * NBUF]
        wsems = bufs_sems[2 * NBUF:]

        wid = lax.axis_index("s") * NUM_CORES + lax.axis_index("c")
        base = wid * per_worker
        row_base = wid * rows_per_worker

        pltpu.sync_copy(idx_hbm.at[pl.ds(base, per_worker)], idx_v)

        def idx_slice(c):
            return idx_v.at[pl.ds(c * CHUNK, CHUNK)]

        def wb_start(b, c):
            for r in range(rows_per_chunk):
                pltpu.async_copy(
                    rbufs[b].at[pl.ds(r * hist, hist), pl.ds(0, EMBED_DIM)],
                    out_hbm.at[row_base + c * rows_per_chunk + r],
                    wsems[b])

        def wb_wait(b, c):
            for r in range(rows_per_chunk):
                pltpu.make_async_copy(
                    rbufs[b].at[pl.ds(r * hist, hist), pl.ds(0, EMBED_DIM)],
                    out_hbm.at[row_base + c * rows_per_chunk + r],
                    wsems[b]).wait()

        for b in range(NBUF - 1):
            pltpu.async_copy(table_hbm.at[idx_slice(b)], rbufs[b], gsems[b])

        @pl.loop(0, n_chunks, step=NBUF)
        def _(k):
            for b in range(NBUF):
                c = k + b
                nxt = c + NBUF - 1
                nb = (b + NBUF - 1) % NBUF
                pltpu.make_async_copy(table_hbm.at[idx_slice(c)],
                                      rbufs[b], gsems[b]).wait()
                wb_start(b, c)

                @pl.when(nxt < n_chunks)
                def _():
                    @pl.when(c >= 1)
                    def _():
                        wb_wait(nb, c - 1)

                    pltpu.async_copy(table_hbm.at[idx_slice(nxt)],
                                     rbufs[nb], gsems[nb])

        for j in range(NBUF):
            c = n_chunks - NBUF + j
            wb_wait(c % NBUF, c)

    return gather_kernel(idx, w128)
